# baseline (speedup 1.0000x reference)
import jax
import jax.numpy as jnp
from jax import lax
from jax.experimental import pallas as pl
from jax.experimental.pallas import tpu as pltpu
from jax.experimental.pallas import tpu_sc as plsc

_VOCAB = 100000
_DIM = 128
_B = 4096
_LOSS_MARGIN = 1.0
_LAMBDA_W = 0.5

_NC = 2
_NS = 16
_NW = _NC * _NS
_ROWS_PER_W = _B // _NW
_CH = 32
_NCH = _ROWS_PER_W // _CH
_L = 16
_D2 = _DIM // 2


def _sc_kernel_body(
    h_x, idx_all, entity_table, relation_table, attrib_table,
    out_partials,
    idx0, idx1,
    he0, ha0, tpe0, tpa0, tne0, tna0, rp0, rn0, hx0,
    he1, ha1, tpe1, tpa1, tne1, tna1, rp1, rn1, hx1,
    acc_v, sem_i0, sem_i1, sem_g0, sem_g1,
):
  wid = lax.axis_index("s") * _NC + lax.axis_index("c")
  chunk0 = wid * _NCH

  idx_v = (idx0, idx1)
  sem_i = (sem_i0, sem_i1)
  sem_g = (sem_g0, sem_g1)
  bufs = (
      (he0, ha0, tpe0, tpa0, tne0, tna0, rp0, rn0, hx0),
      (he1, ha1, tpe1, tpa1, tne1, tna1, rp1, rn1, hx1),
  )

  def stage_idx(slot, ch):
    b0 = (chunk0 + ch) * _CH
    return [
        pltpu.async_copy(idx_all.at[pl.ds(k * _B + b0, _CH)],
                         idx_v[slot].at[k], sem_i[slot])
        for k in range(8)
    ]

  def fire(slot, ch):
    iv = idx_v[slot]
    b = bufs[slot]
    b0 = (chunk0 + ch) * _CH
    return [
        pltpu.async_copy(entity_table.at[iv.at[0]], b[0], sem_g[slot]),
        pltpu.async_copy(attrib_table.at[iv.at[1]], b[1], sem_g[slot]),
        pltpu.async_copy(entity_table.at[iv.at[2]], b[2], sem_g[slot]),
        pltpu.async_copy(attrib_table.at[iv.at[3]], b[3], sem_g[slot]),
        pltpu.async_copy(entity_table.at[iv.at[4]], b[4], sem_g[slot]),
        pltpu.async_copy(attrib_table.at[iv.at[5]], b[5], sem_g[slot]),
        pltpu.async_copy(relation_table.at[iv.at[6]], b[6], sem_g[slot]),
        pltpu.async_copy(relation_table.at[iv.at[7]], b[7], sem_g[slot]),
        pltpu.async_copy(h_x.at[pl.ds(b0, _CH)], b[8], sem_g[slot]),
    ]

  lanes = jnp.arange(_L, dtype=jnp.int32)
  perms = [lanes ^ d for d in (1, 2, 4, 8)]
  margin = jnp.full((_L,), _LOSS_MARGIN, jnp.float32)
  zero = jnp.zeros((_L,), jnp.float32)

  def make_row_body(b):
    he, ha, tpe, tpa, tne, tna, rp, rn, hx = b

    def row_body(i, acc):
      diff = jnp.zeros((_L,), jnp.float32)
      for j in range(_D2 // _L):
        cr = pl.ds(j * _L, _L)
        ci = pl.ds(_D2 + j * _L, _L)
        re_h = he[i, cr] + ha[i, cr] + hx[i, cr]
        im_h = he[i, ci] + ha[i, ci] + hx[i, ci]
        re_tp = tpe[i, cr] + tpa[i, cr]
        im_tp = tpe[i, ci] + tpa[i, ci]
        re_tn = tne[i, cr] + tna[i, cr]
        im_tn = tne[i, ci] + tna[i, ci]
        re_rp = rp[i, cr]
        im_rp = rp[i, ci]
        re_rn = rn[i, cr]
        im_rn = rn[i, ci]
        diff = diff + (
            re_rn * (re_h * re_tn + im_h * im_tn)
            + im_rn * (re_h * im_tn - im_h * re_tn)
            - re_rp * (re_h * re_tp + im_h * im_tp)
            - im_rp * (re_h * im_tp - im_h * re_tp)
        )
      for perm in perms:
        diff = diff + diff.at[perm].get(mode="promise_in_bounds")
      return acc + jnp.maximum(margin + diff, zero)

    return row_body

  d_i = [None, None]
  g = [None, None]
  for d in stage_idx(0, 0):
    d.wait()
  g[0] = fire(0, 0)
  d_i[1] = stage_idx(1, 1)

  acc = jnp.zeros((_L,), jnp.float32)
  for ch in range(_NCH):
    cur = ch % 2
    nxt = 1 - cur
    if ch + 1 < _NCH:
      for d in d_i[nxt]:
        d.wait()
      g[nxt] = fire(nxt, ch + 1)
    for d in g[cur]:
      d.wait()
    if ch + 2 < _NCH:
      d_i[cur] = stage_idx(cur, ch + 2)
    acc = lax.fori_loop(0, _CH, make_row_body(bufs[cur]), acc)

  acc_v[...] = acc
  pltpu.sync_copy(acc_v, out_partials.at[wid])


def kernel(h_x, referents, pos_relations, neg_relations, positive_samples,
           negative_samples, referent_attribs, positive_attribs,
           negative_attribs, entity_table, relation_table, attrib_table):
  mesh = plsc.VectorSubcoreMesh(core_axis_name="c", subcore_axis_name="s")
  row_t = pltpu.VMEM((_CH, _DIM), jnp.float32)
  run = pl.kernel(
      _sc_kernel_body,
      mesh=mesh,
      out_type=jax.ShapeDtypeStruct((_NW, _L), jnp.float32),
      scratch_types=[
          pltpu.VMEM((8, _CH), jnp.int32),
          pltpu.VMEM((8, _CH), jnp.int32),
          row_t, row_t, row_t, row_t, row_t, row_t, row_t, row_t, row_t,
          row_t, row_t, row_t, row_t, row_t, row_t, row_t, row_t, row_t,
          pltpu.VMEM((_L,), jnp.float32),
          pltpu.SemaphoreType.DMA,
          pltpu.SemaphoreType.DMA,
          pltpu.SemaphoreType.DMA,
          pltpu.SemaphoreType.DMA,
      ],
  )
  idx_all = jnp.concatenate([
      referents, referent_attribs, positive_samples, positive_attribs,
      negative_samples, negative_attribs, pos_relations, neg_relations,
  ]).astype(jnp.int32)
  partials = run(h_x, idx_all, entity_table, relation_table, attrib_table)
  loss = jnp.sum(partials[:, 0]) * (_LAMBDA_W / _B)
  return (loss, h_x)


# speedup vs baseline: 1.0881x; 1.0881x over previous
import jax
import jax.numpy as jnp
from jax import lax
from jax.experimental import pallas as pl
from jax.experimental.pallas import tpu as pltpu
from jax.experimental.pallas import tpu_sc as plsc

_VOCAB = 100000
_DIM = 128
_B = 4096
_LOSS_MARGIN = 1.0
_LAMBDA_W = 0.5

_NC = 2
_NS = 16
_NW = _NC * _NS
_ROWS_PER_W = _B // _NW
_CH = 32
_NCH = _ROWS_PER_W // _CH
_L = 16
_D2 = _DIM // 2


def _sc_kernel_body(
    h_x, referents, referent_attribs, positive_samples, positive_attribs,
    negative_samples, negative_attribs, pos_relations, neg_relations,
    entity_table, relation_table, attrib_table,
    out_partials,
    idx_v, hx_all,
    he0, ha0, tpe0, tpa0, tne0, tna0, rp0, rn0,
    he1, ha1, tpe1, tpa1, tne1, tna1, rp1, rn1,
    acc_v, sem_i, sem_h, sem_g0, sem_g1,
):
  wid = lax.axis_index("s") * _NC + lax.axis_index("c")
  base = wid * _ROWS_PER_W

  sem_g = (sem_g0, sem_g1)
  bufs = (
      (he0, ha0, tpe0, tpa0, tne0, tna0, rp0, rn0),
      (he1, ha1, tpe1, tpa1, tne1, tna1, rp1, rn1),
  )

  idx_srcs = (referents, referent_attribs, positive_samples,
              positive_attribs, negative_samples, negative_attribs,
              pos_relations, neg_relations)

  def fire(slot, ch):
    b = bufs[slot]
    r0 = ch * _CH

    def iv(k):
      return idx_v.at[k, pl.ds(r0, _CH)]

    return [
        pltpu.async_copy(entity_table.at[iv(0)], b[0], sem_g[slot]),
        pltpu.async_copy(attrib_table.at[iv(1)], b[1], sem_g[slot]),
        pltpu.async_copy(entity_table.at[iv(2)], b[2], sem_g[slot]),
        pltpu.async_copy(attrib_table.at[iv(3)], b[3], sem_g[slot]),
        pltpu.async_copy(entity_table.at[iv(4)], b[4], sem_g[slot]),
        pltpu.async_copy(attrib_table.at[iv(5)], b[5], sem_g[slot]),
        pltpu.async_copy(relation_table.at[iv(6)], b[6], sem_g[slot]),
        pltpu.async_copy(relation_table.at[iv(7)], b[7], sem_g[slot]),
    ]

  lanes = jnp.arange(_L, dtype=jnp.int32)
  perms = [lanes ^ d for d in (1, 2, 4, 8)]
  margin = jnp.full((_L,), _LOSS_MARGIN, jnp.float32)
  zero = jnp.zeros((_L,), jnp.float32)

  def make_row_body(b, r0):
    he, ha, tpe, tpa, tne, tna, rp, rn = b

    def row_body(i, acc):
      diff = jnp.zeros((_L,), jnp.float32)
      for j in range(_D2 // _L):
        cr = pl.ds(j * _L, _L)
        ci = pl.ds(_D2 + j * _L, _L)
        re_h = he[i, cr] + ha[i, cr] + hx_all[r0 + i, cr]
        im_h = he[i, ci] + ha[i, ci] + hx_all[r0 + i, ci]
        re_tp = tpe[i, cr] + tpa[i, cr]
        im_tp = tpe[i, ci] + tpa[i, ci]
        re_tn = tne[i, cr] + tna[i, cr]
        im_tn = tne[i, ci] + tna[i, ci]
        re_rp = rp[i, cr]
        im_rp = rp[i, ci]
        re_rn = rn[i, cr]
        im_rn = rn[i, ci]
        diff = diff + (
            re_rn * (re_h * re_tn + im_h * im_tn)
            + im_rn * (re_h * im_tn - im_h * re_tn)
            - re_rp * (re_h * re_tp + im_h * im_tp)
            - im_rp * (re_h * im_tp - im_h * re_tp)
        )
      for perm in perms:
        diff = diff + diff.at[perm].get(mode="promise_in_bounds")
      return acc + jnp.maximum(margin + diff, zero)

    return row_body

  d_i = [
      pltpu.async_copy(src.at[pl.ds(base, _ROWS_PER_W)], idx_v.at[k], sem_i)
      for k, src in enumerate(idx_srcs)
  ]
  d_h = pltpu.async_copy(h_x.at[pl.ds(base, _ROWS_PER_W)], hx_all, sem_h)
  for d in d_i:
    d.wait()
  g = [fire(0, 0), fire(1, 1)]
  d_h.wait()

  acc = jnp.zeros((_L,), jnp.float32)
  for ch in range(_NCH):
    cur = ch % 2
    for d in g[cur]:
      d.wait()
    acc = lax.fori_loop(0, _CH, make_row_body(bufs[cur], ch * _CH), acc)
    if ch + 2 < _NCH:
      g[cur] = fire(cur, ch + 2)

  acc_v[...] = acc
  pltpu.sync_copy(acc_v, out_partials.at[wid])


def kernel(h_x, referents, pos_relations, neg_relations, positive_samples,
           negative_samples, referent_attribs, positive_attribs,
           negative_attribs, entity_table, relation_table, attrib_table):
  mesh = plsc.VectorSubcoreMesh(core_axis_name="c", subcore_axis_name="s")
  row_t = pltpu.VMEM((_CH, _DIM), jnp.float32)
  run = pl.kernel(
      _sc_kernel_body,
      mesh=mesh,
      out_type=jax.ShapeDtypeStruct((_NW, _L), jnp.float32),
      scratch_types=[
          pltpu.VMEM((8, _ROWS_PER_W), jnp.int32),
          pltpu.VMEM((_ROWS_PER_W, _DIM), jnp.float32),
          row_t, row_t, row_t, row_t, row_t, row_t, row_t, row_t,
          row_t, row_t, row_t, row_t, row_t, row_t, row_t, row_t,
          pltpu.VMEM((_L,), jnp.float32),
          pltpu.SemaphoreType.DMA,
          pltpu.SemaphoreType.DMA,
          pltpu.SemaphoreType.DMA,
          pltpu.SemaphoreType.DMA,
      ],
  )
  partials = run(
      h_x, referents.astype(jnp.int32), referent_attribs.astype(jnp.int32),
      positive_samples.astype(jnp.int32), positive_attribs.astype(jnp.int32),
      negative_samples.astype(jnp.int32), negative_attribs.astype(jnp.int32),
      pos_relations.astype(jnp.int32), neg_relations.astype(jnp.int32),
      entity_table, relation_table, attrib_table)
  loss = jnp.sum(partials[:, 0]) * (_LAMBDA_W / _B)
  return (loss, h_x)
